# baseline (speedup 1.0000x reference)
import functools

import jax
import jax.numpy as jnp
from jax import lax
from jax.experimental import pallas as pl
from jax.experimental.pallas import tpu as pltpu
from jax.experimental.pallas import tpu_sc as plsc

_BS = 4
_D0 = 128
_D1 = 8192
_K = 8
_ROWS = _BS * _D0
_L = 16
_CHUNKS = _D1 // _L
_NW = 32
_RPW = _ROWS // _NW

_NEG = float("-inf")
_POS = float("inf")

def _bmax(x, lane):
    del lane
    return plsc.cummax(lax.rev(plsc.cummax(x), (0,)))


def _sc_body(inp_hbm, gn_hbm, out_hbm, inp_v, gn_v, pbuf, candv, candi,
             outb, nbuf, sem):
    cid = lax.axis_index("c")
    sid = lax.axis_index("s")
    wid = sid * 2 + cid
    row0 = wid * _RPW
    lane = lax.iota(jnp.int32, _L)

    def row_body(r, _carry):
        row = row0 + r
        i = lax.rem(row, _D0)
        pltpu.sync_copy(inp_hbm.at[i], inp_v)
        pltpu.sync_copy(gn_hbm.at[row], gn_v)

        def p1(t, M):
            v = inp_v[pl.ds(t * _L, _L)] + gn_v[pl.ds(t * _L, _L)]
            pbuf[pl.ds(t * _L, _L)] = v
            return jnp.maximum(M, v)

        M = lax.fori_loop(0, _CHUNKS, p1, jnp.full((_L,), _NEG, jnp.float32))

        T = -_bmax(-M, lane)

        def p2(t, PC):
            v = pbuf[pl.ds(t * _L, _L)]
            msk = v >= T
            pos = PC * _L + lane
            idxv = t * _L + lane
            plsc.store_scatter(candv, [pos], v, mask=msk)
            plsc.store_scatter(candi, [pos], idxv, mask=msk)
            return PC + msk.astype(jnp.int32)

        PC = lax.fori_loop(0, _CHUNKS, p2, jnp.zeros((_L,), jnp.int32))
        nbuf[...] = _bmax(PC, lane)
        nch = nbuf[...][0]

        sel = []
        m = jnp.full((_L,), _POS, jnp.float32)
        bi = jnp.zeros((_L,), jnp.int32)
        for _j in range(_K):
            def scan(k, carry):
                bv, bic = carry
                v = candv[pl.ds(k * _L, _L)]
                ci = candi[pl.ds(k * _L, _L)]
                valid = k < PC
                excl = (v > m) | ((v == m) & (ci <= bi))
                vv = jnp.where(valid & ~excl, v, _NEG)
                upd = (vv > bv) | ((vv == bv) & (ci < bic))
                return jnp.where(upd, vv, bv), jnp.where(upd, ci, bic)

            bv, bic = lax.fori_loop(
                0, nch, scan,
                (jnp.full((_L,), _NEG, jnp.float32),
                 jnp.full((_L,), _D1, jnp.int32)))
            m = _bmax(bv, lane)
            bi = -_bmax(-jnp.where(bv == m, bic, _D1), lane)
            sel.append(bi)

        vv = jnp.full((_L,), _D1, jnp.int32)
        for j in range(_K):
            vv = jnp.where(lane == j, sel[j], vv)
        srt, _ = plsc.sort_key_val(vv, vv)
        plsc.store_compressed(outb.at[pl.ds(r * _K, _L)], srt,
                              mask=lane < _K)
        return 0

    lax.fori_loop(0, _RPW, row_body, 0)
    pltpu.sync_copy(outb.at[pl.ds(0, _RPW * _K)],
                    out_hbm.at[pl.ds(row0 * _K, _RPW * _K)])


@jax.jit
def _sc_topk(inp, gn_flat):
    mesh = plsc.VectorSubcoreMesh(core_axis_name="c", subcore_axis_name="s")
    return pl.kernel(
        _sc_body,
        mesh=mesh,
        out_type=jax.ShapeDtypeStruct((_ROWS * _K,), jnp.int32),
        compiler_params=pltpu.CompilerParams(needs_layout_passes=False),
        scratch_types=[
            pltpu.VMEM((_D1,), jnp.float32),
            pltpu.VMEM((_D1,), jnp.float32),
            pltpu.VMEM((_D1,), jnp.float32),
            pltpu.VMEM((_D1,), jnp.float32),
            pltpu.VMEM((_D1,), jnp.int32),
            pltpu.VMEM((_RPW * _K + _L,), jnp.int32),
            pltpu.VMEM((_L,), jnp.int32),
            pltpu.SemaphoreType.DMA,
        ],
    )(inp, gn_flat)


def _tc_body(idx_ref, out_ref, *, rows):
    col = jax.lax.broadcasted_iota(jnp.int32, (rows, _D1), 1)
    for j in range(_K):
        out_ref[0, :, j, :] = (col == idx_ref[:, j:j + 1]).astype(jnp.float32)


@functools.partial(jax.jit, static_argnames=("rows",))
def _tc_onehot(idx, rows=64):
    grid = (_BS, _D0 // rows)
    blocks_per_b = _D0 // rows
    return pl.pallas_call(
        functools.partial(_tc_body, rows=rows),
        grid=grid,
        in_specs=[
            pl.BlockSpec((rows, _K), lambda b, i: (b * blocks_per_b + i, 0)),
        ],
        out_specs=pl.BlockSpec((1, rows, _K, _D1), lambda b, i: (b, i, 0, 0)),
        out_shape=jax.ShapeDtypeStruct((_BS, _D0, _K, _D1), jnp.float32),
    )(idx)


def kernel(inp, GN):
    gn_flat = GN.reshape(_ROWS, _D1)
    idx = _sc_topk(inp, gn_flat).reshape(_ROWS, _K)
    return _tc_onehot(idx)
